# baseline (speedup 1.0000x reference)
import functools

import jax
import jax.numpy as jnp
from jax.experimental import pallas as pl
from jax.experimental.pallas import tpu as pltpu

H = 1024
L = 4
DS = [1024, 512, 256, 128]
DSUM = sum(DS)
PCOLS = 128
NEG = -1e30


def _fused_body(x_ref, wp1_ref, bp1_ref, wp2_ref, bp2_ref,
                wc1_ref, bc1_ref, wc2_ref, colmap_ref, cb2_ref, o_ref):
    x = x_ref[...]
    hp = jnp.maximum(
        jax.lax.dot(x, wp1_ref[...], precision=jax.lax.Precision.DEFAULT)
        + bp1_ref[...], 0.0)
    logits = jax.lax.dot(hp, wp2_ref[...],
                         precision=jax.lax.Precision.DEFAULT) + bp2_ref[...]
    m = logits.shape[0]
    maxv = jnp.max(logits, axis=1, keepdims=True)
    lane = jax.lax.broadcasted_iota(jnp.int32, (m, PCOLS), 1)
    levels = jnp.min(jnp.where(logits == maxv, lane, PCOLS),
                     axis=1, keepdims=True)
    hc = jnp.maximum(
        jax.lax.dot(x.astype(jnp.bfloat16), wc1_ref[...],
                    precision=jax.lax.Precision.DEFAULT,
                    preferred_element_type=jnp.float32)
        + bc1_ref[...], 0.0)
    mask = levels == colmap_ref[...]
    hcm = jnp.where(mask, hc, 0.0).astype(jnp.bfloat16)
    out = jax.lax.dot(hcm, wc2_ref[...],
                      precision=jax.lax.Precision.DEFAULT,
                      preferred_element_type=jnp.float32)
    for l in range(L):
        out = out + jnp.where(levels == l, cb2_ref[l][None, :], 0.0)
    o_ref[...] = out


@functools.partial(jax.jit, static_argnames=("bm",))
def _run(x, wp1, bp1, wp2, bp2, wc1, bc1, wc2, colmap, cb2, bm=512):
    n = x.shape[0]
    grid = (n // bm,)
    full = lambda shape: pl.BlockSpec(shape, lambda i: (0,) * len(shape))
    return pl.pallas_call(
        _fused_body,
        grid=grid,
        in_specs=[
            pl.BlockSpec((bm, H), lambda i: (i, 0)),
            full((H, H)), full((1, H)),
            full((H, PCOLS)), full((1, PCOLS)),
            full((H, DSUM)), full((1, DSUM)),
            full((DSUM, H)), full((1, DSUM)),
            full((L, H)),
        ],
        out_specs=pl.BlockSpec((bm, H), lambda i: (i, 0)),
        out_shape=jax.ShapeDtypeStruct((n, H), jnp.float32),
        compiler_params=pltpu.CompilerParams(
            dimension_semantics=("arbitrary",),
        ),
    )(x, wp1, bp1, wp2, bp2, wc1, bc1, wc2, colmap, cb2)


def kernel(keys, values,
           pw1_0, pb1_0, pw2_0, pb2_0,
           pw1_1, pb1_1, pw2_1, pb2_1,
           pw1_2, pb1_2, pw2_2, pb2_2,
           pw1_3, pb1_3, pw2_3, pb2_3,
           cw1_0, cb1_0, cw2_0, cb2_0,
           cw1_1, cb1_1, cw2_1, cb2_1,
           cw1_2, cb1_2, cw2_2, cb2_2,
           cw1_3, cb1_3, cw2_3, cb2_3):
    pw1 = [pw1_0, pw1_1, pw1_2, pw1_3]
    pb1 = [pb1_0, pb1_1, pb1_2, pb1_3]
    pw2 = [pw2_0, pw2_1, pw2_2, pw2_3]
    pb2 = [pb2_0, pb2_1, pb2_2, pb2_3]
    cw1 = [cw1_0, cw1_1, cw1_2, cw1_3]
    cb1 = [cb1_0, cb1_1, cb1_2, cb1_3]
    cw2 = [cw2_0, cw2_1, cw2_2, cw2_3]
    cb2 = [cb2_0, cb2_1, cb2_2, cb2_3]

    wp1 = jnp.concatenate(pw1, axis=1)
    bp1 = jnp.concatenate(pb1)[None, :]
    wp2 = jnp.zeros((H, PCOLS), jnp.float32)
    for l in range(L):
        wp2 = wp2.at[l * (H // L):(l + 1) * (H // L), l].set(pw2[l][:, 0])
    bp2 = jnp.full((PCOLS,), NEG, jnp.float32)
    bp2 = bp2.at[:L].set(jnp.concatenate(pb2))[None, :]

    wc1 = jnp.concatenate(cw1, axis=1).astype(jnp.bfloat16)
    bc1 = jnp.concatenate(cb1)[None, :]
    wc2 = jnp.concatenate(cw2, axis=0).astype(jnp.bfloat16)
    colmap = jnp.concatenate(
        [jnp.full((d,), l, jnp.int32) for l, d in enumerate(DS)])[None, :]
    cb2s = jnp.stack(cb2)

    b, s, _ = keys.shape
    x = jnp.concatenate(
        [keys.reshape(b * s, H), values.reshape(b * s, H)], axis=0)
    out = _run(x, wp1, bp1, wp2, bp2, wc1, bc1, wc2, colmap, cb2s)
    ck = out[:b * s].reshape(b, s, H)
    cv = out[b * s:].reshape(b, s, H)
    return (ck, cv)


# speedup vs baseline: 1.0085x; 1.0085x over previous
import functools

import jax
import jax.numpy as jnp
from jax.experimental import pallas as pl
from jax.experimental.pallas import tpu as pltpu

H = 1024
L = 4
DS = [1024, 512, 256, 128]
DSUM = sum(DS)
PCOLS = 128
NEG = -1e30


def _fused_body(x_ref, wp1_ref, bp1_ref, wp2_ref, bp2_ref,
                wc1_ref, bc1_ref, wc2_ref, colmap_ref, cb2_ref, o_ref):
    x = x_ref[...]
    hp = jnp.maximum(
        jax.lax.dot(x, wp1_ref[...], precision=jax.lax.Precision.DEFAULT)
        + bp1_ref[...], 0.0)
    logits = jax.lax.dot(hp, wp2_ref[...],
                         precision=jax.lax.Precision.DEFAULT) + bp2_ref[...]
    m = logits.shape[0]
    maxv = jnp.max(logits, axis=1, keepdims=True)
    lane = jax.lax.broadcasted_iota(jnp.int32, (m, PCOLS), 1)
    levels = jnp.min(jnp.where(logits == maxv, lane, PCOLS),
                     axis=1, keepdims=True)
    hc = jnp.maximum(
        jax.lax.dot(x.astype(jnp.bfloat16), wc1_ref[...],
                    precision=jax.lax.Precision.DEFAULT,
                    preferred_element_type=jnp.float32)
        + bc1_ref[...], 0.0)
    mask = levels == colmap_ref[...]
    hcm = jnp.where(mask, hc, 0.0).astype(jnp.bfloat16)
    out = jax.lax.dot(hcm, wc2_ref[...],
                      precision=jax.lax.Precision.DEFAULT,
                      preferred_element_type=jnp.float32)
    for l in range(L):
        out = out + jnp.where(levels == l, cb2_ref[l][None, :], 0.0)
    o_ref[...] = out


@functools.partial(jax.jit, static_argnames=("bm",))
def _run(x, wp1, bp1, wp2, bp2, wc1, bc1, wc2, colmap, cb2, bm=1024):
    n = x.shape[0]
    grid = (n // bm,)
    full = lambda shape: pl.BlockSpec(shape, lambda i: (0,) * len(shape))
    return pl.pallas_call(
        _fused_body,
        grid=grid,
        in_specs=[
            pl.BlockSpec((bm, H), lambda i: (i, 0)),
            full((H, H)), full((1, H)),
            full((H, PCOLS)), full((1, PCOLS)),
            full((H, DSUM)), full((1, DSUM)),
            full((DSUM, H)), full((1, DSUM)),
            full((L, H)),
        ],
        out_specs=pl.BlockSpec((bm, H), lambda i: (i, 0)),
        out_shape=jax.ShapeDtypeStruct((n, H), jnp.float32),
        compiler_params=pltpu.CompilerParams(
            dimension_semantics=("arbitrary",),
        ),
    )(x, wp1, bp1, wp2, bp2, wc1, bc1, wc2, colmap, cb2)


def kernel(keys, values,
           pw1_0, pb1_0, pw2_0, pb2_0,
           pw1_1, pb1_1, pw2_1, pb2_1,
           pw1_2, pb1_2, pw2_2, pb2_2,
           pw1_3, pb1_3, pw2_3, pb2_3,
           cw1_0, cb1_0, cw2_0, cb2_0,
           cw1_1, cb1_1, cw2_1, cb2_1,
           cw1_2, cb1_2, cw2_2, cb2_2,
           cw1_3, cb1_3, cw2_3, cb2_3):
    pw1 = [pw1_0, pw1_1, pw1_2, pw1_3]
    pb1 = [pb1_0, pb1_1, pb1_2, pb1_3]
    pw2 = [pw2_0, pw2_1, pw2_2, pw2_3]
    pb2 = [pb2_0, pb2_1, pb2_2, pb2_3]
    cw1 = [cw1_0, cw1_1, cw1_2, cw1_3]
    cb1 = [cb1_0, cb1_1, cb1_2, cb1_3]
    cw2 = [cw2_0, cw2_1, cw2_2, cw2_3]
    cb2 = [cb2_0, cb2_1, cb2_2, cb2_3]

    wp1 = jnp.concatenate(pw1, axis=1)
    bp1 = jnp.concatenate(pb1)[None, :]
    wp2 = jnp.zeros((H, PCOLS), jnp.float32)
    for l in range(L):
        wp2 = wp2.at[l * (H // L):(l + 1) * (H // L), l].set(pw2[l][:, 0])
    bp2 = jnp.full((PCOLS,), NEG, jnp.float32)
    bp2 = bp2.at[:L].set(jnp.concatenate(pb2))[None, :]

    wc1 = jnp.concatenate(cw1, axis=1).astype(jnp.bfloat16)
    bc1 = jnp.concatenate(cb1)[None, :]
    wc2 = jnp.concatenate(cw2, axis=0).astype(jnp.bfloat16)
    colmap = jnp.concatenate(
        [jnp.full((d,), l, jnp.int32) for l, d in enumerate(DS)])[None, :]
    cb2s = jnp.stack(cb2)

    b, s, _ = keys.shape
    x = jnp.concatenate(
        [keys.reshape(b * s, H), values.reshape(b * s, H)], axis=0)
    out = _run(x, wp1, bp1, wp2, bp2, wc1, bc1, wc2, colmap, cb2s)
    ck = out[:b * s].reshape(b, s, H)
    cv = out[b * s:].reshape(b, s, H)
    return (ck, cv)
